# baseline (speedup 1.0000x reference)
import functools

import jax
import jax.numpy as jnp
import numpy as np
from jax import lax
from jax.experimental import pallas as pl
from jax.experimental.pallas import tpu as pltpu
from jax.experimental.pallas import tpu_sc as plsc

_N, _DEG, _F, _H, _L, _B = 10000, 32, 128, 128, 64, 1024
_NC, _NS = 2, 16
_NW = _NC * _NS
_NPAD = 10240
_RW = _NPAD // _NW
_NB = 4
_BLK = _RW // _NB
_IDXR = _NPAD * _DEG // 128
_BW = _B // _NW
_NBUF = 4

_mesh = plsc.VectorSubcoreMesh(core_axis_name="c", subcore_axis_name="s")


def _acc_rows(rows, row0, nrows):
    def step(k, accs):
        base = row0 + k * 8
        for dd in range(8):
            r = base + dd
            accs = tuple(accs[v] + rows[r, pl.ds(16 * v, 16)] for v in range(8))
        return accs
    zeros = tuple(jnp.zeros((16,), jnp.float32) for _ in range(8))
    return lax.fori_loop(0, nrows // 8, step, zeros)


def _acc_rows_pk(rows, nrows):
    mask = jnp.int32(-65536)

    def step(k, accs):
        base = k * 8
        for dd in range(8):
            r = base + dd
            na = []
            for v in range(4):
                w = rows[r, pl.ds(16 * v, 16)]
                lo = plsc.bitcast(jnp.left_shift(w, 16), jnp.float32)
                hi = plsc.bitcast(jnp.bitwise_and(w, mask), jnp.float32)
                na.append(accs[2 * v] + lo)
                na.append(accs[2 * v + 1] + hi)
            accs = tuple(na)
        return accs

    zeros = tuple(jnp.zeros((16,), jnp.float32) for _ in range(8))
    return lax.fori_loop(0, nrows // 8, step, zeros)


_PERM = np.empty(128, np.int32)
for _v in range(4):
    for _k in range(16):
        _PERM[32 * _v + _k] = 32 * _v + 2 * _k
        _PERM[32 * _v + 16 + _k] = 32 * _v + 2 * _k + 1


_ABUF = 8


def _nmean_body(adj_pad, feats, out, idx_all, rows0, rows1, rows2, rows3,
                rows4, rows5, rows6, rows7,
                out_all, sem0, sem1, sem2, sem3, sem4, sem5, sem6, sem7):
    wid = lax.axis_index("s") * _NC + lax.axis_index("c")
    pltpu.sync_copy(adj_pad.at[pl.ds(wid * _RW, _RW)], idx_all)
    bufs = (rows0, rows1, rows2, rows3, rows4, rows5, rows6, rows7)
    sems = (sem0, sem1, sem2, sem3, sem4, sem5, sem6, sem7)
    for p in range(_ABUF):
        pltpu.async_copy(feats.at[idx_all.at[p]], bufs[p], sems[p])

    def do_node(t, rows, sem):
        pltpu.make_async_copy(feats.at[pl.ds(0, _DEG)], rows, sem).wait()
        accs = _acc_rows_pk(rows, _DEG)
        for v in range(4):
            out_all[t, pl.ds(32 * v, 16)] = accs[2 * v] * (1.0 / _DEG)
            out_all[t, pl.ds(32 * v + 16, 16)] = accs[2 * v + 1] * (1.0 / _DEG)

        @pl.when(t < _RW - _ABUF)
        def _():
            pltpu.async_copy(feats.at[idx_all.at[t + _ABUF]], rows, sem)

    def loop(i, carry):
        for p in range(_ABUF):
            do_node(_ABUF * i + p, bufs[p], sems[p])
        return carry

    lax.fori_loop(0, _RW // _ABUF, loop, 0)
    pltpu.sync_copy(out_all, out.at[pl.ds(wid * _RW, _RW)])


_nmean_call = functools.partial(
    pl.kernel,
    out_type=jax.ShapeDtypeStruct((_NPAD, _F), jnp.float32),
    mesh=_mesh,
    compiler_params=pltpu.CompilerParams(needs_layout_passes=False,
                                         use_tc_tiling_on_sc=False),
    scratch_types=[
        pltpu.VMEM((_RW, _DEG), jnp.int32),
    ] + [pltpu.VMEM((_DEG, _F // 2), jnp.int32)] * _ABUF + [
        pltpu.VMEM((_RW, _F), jnp.float32),
    ] + [pltpu.SemaphoreType.DMA] * _ABUF,
)(_nmean_body)


def _layer2_body(batch, adj, h, node_out, neigh_out,
                 bidx, nidx, nodes_v,
                 nr0, nr1, nr2, nr3, out_v,
                 sa, sn, s0, s1, s2, s3):
    wid = lax.axis_index("s") * _NC + lax.axis_index("c")
    pltpu.sync_copy(batch.at[pl.ds(wid * _BW, _BW)], bidx)
    pltpu.async_copy(h.at[bidx], nodes_v, sn)
    for g in range(_BW // 16):
        bv = bidx[pl.ds(g * 16, 16)]
        for l in range(16):
            b = g * 16 + l
            pltpu.async_copy(adj.at[pl.ds(bv[l], 1)],
                             nidx.at[pl.ds(b, 1)], sa)
    pltpu.make_async_copy(adj.at[pl.ds(0, _BW)], nidx, sa).wait()

    bufs = (nr0, nr1, nr2, nr3)
    sems = (s0, s1, s2, s3)
    for p in range(_NBUF):
        pltpu.async_copy(h.at[nidx.at[p]], bufs[p], sems[p])

    def do_elem(b, buf, sem):
        pltpu.make_async_copy(h.at[pl.ds(0, _DEG)], buf, sem).wait()
        accs = _acc_rows_pk(buf, _DEG)
        for v in range(4):
            out_v[b, pl.ds(32 * v, 16)] = accs[2 * v] * (1.0 / _DEG)
            out_v[b, pl.ds(32 * v + 16, 16)] = accs[2 * v + 1] * (1.0 / _DEG)

        @pl.when(b < _BW - _NBUF)
        def _():
            pltpu.async_copy(h.at[nidx.at[b + _NBUF]], buf, sem)

    def loop(i, carry):
        for p in range(_NBUF):
            do_elem(i * _NBUF + p, bufs[p], sems[p])
        return carry

    lax.fori_loop(0, _BW // _NBUF, loop, 0)
    pltpu.make_async_copy(h.at[pl.ds(0, _BW)], nodes_v, sn).wait()
    pltpu.sync_copy(nodes_v, node_out.at[pl.ds(wid * _BW, _BW)])
    pltpu.sync_copy(out_v, neigh_out.at[pl.ds(wid * _BW, _BW)])


_layer2_call = functools.partial(
    pl.kernel,
    out_type=(
        jax.ShapeDtypeStruct((_B, _H // 2), jnp.int32),
        jax.ShapeDtypeStruct((_B, _H), jnp.float32),
    ),
    mesh=_mesh,
    compiler_params=pltpu.CompilerParams(needs_layout_passes=False,
                                         use_tc_tiling_on_sc=False),
    scratch_types=[
        pltpu.VMEM((_BW,), jnp.int32),
        pltpu.VMEM((_BW, _DEG), jnp.int32),
        pltpu.VMEM((_BW, _H // 2), jnp.int32),
        pltpu.VMEM((_DEG, _H // 2), jnp.int32),
        pltpu.VMEM((_DEG, _H // 2), jnp.int32),
        pltpu.VMEM((_DEG, _H // 2), jnp.int32),
        pltpu.VMEM((_DEG, _H // 2), jnp.int32),
        pltpu.VMEM((_BW, _H), jnp.float32),
        pltpu.SemaphoreType.DMA,
        pltpu.SemaphoreType.DMA,
        pltpu.SemaphoreType.DMA,
        pltpu.SemaphoreType.DMA,
        pltpu.SemaphoreType.DMA,
        pltpu.SemaphoreType.DMA,
    ],
)(_layer2_body)


def _m1_body(x_ref, m_ref, wa_ref, wb_ref, b_ref, o_ref):
    o_ref[...] = jnp.maximum(
        jnp.dot(x_ref[...], wa_ref[...], preferred_element_type=jnp.float32)
        + jnp.dot(m_ref[...], wb_ref[...], preferred_element_type=jnp.float32)
        + b_ref[...],
        0.0,
    ).astype(jnp.bfloat16)


def _m2_body(x_ref, m_ref, wa_ref, wb_ref, b_ref, o_ref):
    o_ref[...] = (
        jnp.dot(x_ref[...].astype(jnp.float32), wa_ref[...],
                preferred_element_type=jnp.float32)
        + jnp.dot(m_ref[...], wb_ref[...], preferred_element_type=jnp.float32)
        + b_ref[...]
    )


def _m1(feats_pad, nmean, w1aT, w1bT, b1):
    blk = 1024
    return pl.pallas_call(
        _m1_body,
        grid=(_NPAD // blk,),
        in_specs=[
            pl.BlockSpec((blk, _F), lambda i: (i, 0)),
            pl.BlockSpec((blk, _F), lambda i: (i, 0)),
            pl.BlockSpec((_F, _H), lambda i: (0, 0)),
            pl.BlockSpec((_F, _H), lambda i: (0, 0)),
            pl.BlockSpec((1, _H), lambda i: (0, 0)),
        ],
        out_specs=pl.BlockSpec((blk, _H), lambda i: (i, 0)),
        out_shape=jax.ShapeDtypeStruct((_NPAD, _H), jnp.bfloat16),
    )(feats_pad, nmean, w1aT, w1bT, b1)


def _m2(node_emb, neigh_emb, w2aT, w2bT, b2):
    return pl.pallas_call(
        _m2_body,
        out_shape=jax.ShapeDtypeStruct((_B, _L), jnp.float32),
    )(node_emb, neigh_emb, w2aT, w2bT, b2)


def kernel(adj_list, feats, batch, W1, b1, W2, b2):
    adj_pad = jnp.concatenate([adj_list, adj_list[: _NPAD - _N]], axis=0)
    feats_pad = jnp.pad(feats, ((0, _NPAD - _N), (0, 0)))
    feats_pk = lax.bitcast_convert_type(
        feats_pad.astype(jnp.bfloat16).reshape(_NPAD, _F // 2, 2), jnp.int32)
    w1aT = W1[:, :_F].T
    w1bT_p = W1[:, _F:].T[_PERM, :]
    w2aT = W2[:, :_H].T
    w2bT_p = W2[:, _H:].T[_PERM, :]

    nmean_p = _nmean_call(adj_pad, feats_pk)
    h_all = _m1(feats_pad, nmean_p, w1aT, w1bT_p, b1[None, :])
    h_pk = lax.bitcast_convert_type(
        h_all.reshape(_NPAD, _H // 2, 2), jnp.int32)
    node_pk, neigh_emb_p = _layer2_call(batch, adj_list, h_pk)
    node_emb = lax.bitcast_convert_type(
        node_pk, jnp.bfloat16).reshape(_B, _H)
    return _m2(node_emb, neigh_emb_p, w2aT, w2bT_p, b2[None, :])


# speedup vs baseline: 1.3448x; 1.3448x over previous
import functools

import jax
import jax.numpy as jnp
from jax import lax
from jax.experimental import pallas as pl
from jax.experimental.pallas import tpu as pltpu
from jax.experimental.pallas import tpu_sc as plsc

_N, _DEG, _F, _H, _L, _B = 10000, 32, 128, 128, 64, 1024
_NC, _NS = 2, 16
_NW = _NC * _NS
_NPAD = 10240
_RW = _NPAD // _NW
_NB = 4
_BLK = _RW // _NB
_IDXR = _NPAD * _DEG // 128
_BW = _B // _NW
_NBUF = 4

_mesh = plsc.VectorSubcoreMesh(core_axis_name="c", subcore_axis_name="s")


def _acc_rows(rows, row0, nrows):
    def step(k, accs):
        base = row0 + k * 8
        for dd in range(8):
            r = base + dd
            accs = tuple(accs[v] + rows[r, pl.ds(16 * v, 16)] for v in range(8))
        return accs
    zeros = tuple(jnp.zeros((16,), jnp.float32) for _ in range(8))
    return lax.fori_loop(0, nrows // 8, step, zeros)


_ABUF = 8


def _nmean_body(adj_pad, feats, out, idx_all, rows0, rows1, rows2, rows3,
                rows4, rows5, rows6, rows7,
                out_all, sem0, sem1, sem2, sem3, sem4, sem5, sem6, sem7):
    wid = lax.axis_index("s") * _NC + lax.axis_index("c")
    pltpu.sync_copy(adj_pad.at[pl.ds(wid * _RW, _RW)], idx_all)
    bufs = (rows0, rows1, rows2, rows3, rows4, rows5, rows6, rows7)
    sems = (sem0, sem1, sem2, sem3, sem4, sem5, sem6, sem7)
    for p in range(_ABUF):
        pltpu.async_copy(feats.at[idx_all.at[p]], bufs[p], sems[p])

    def do_node(t, rows, sem):
        pltpu.make_async_copy(feats.at[pl.ds(0, _DEG)], rows, sem).wait()
        accs = _acc_rows(rows, 0, _DEG)
        for v in range(8):
            out_all[t, pl.ds(16 * v, 16)] = accs[v] * (1.0 / _DEG)

        @pl.when(t < _RW - _ABUF)
        def _():
            pltpu.async_copy(feats.at[idx_all.at[t + _ABUF]], rows, sem)

    def loop(i, carry):
        for p in range(_ABUF):
            do_node(_ABUF * i + p, bufs[p], sems[p])
        return carry

    lax.fori_loop(0, _RW // _ABUF, loop, 0)
    pltpu.sync_copy(out_all, out.at[pl.ds(wid * _RW, _RW)])


_nmean_call = functools.partial(
    pl.kernel,
    out_type=jax.ShapeDtypeStruct((_NPAD, _F), jnp.float32),
    mesh=_mesh,
    scratch_types=[
        pltpu.VMEM((_RW, _DEG), jnp.int32),
    ] + [pltpu.VMEM((_DEG, _F), jnp.float32)] * _ABUF + [
        pltpu.VMEM((_RW, _F), jnp.float32),
    ] + [pltpu.SemaphoreType.DMA] * _ABUF,
)(_nmean_body)


def _layer2_body(batch, adj, h, node_out, neigh_out,
                 bidx, nidx, nodes_v,
                 nr0, nr1, nr2, nr3, out_v,
                 sa, sn, s0, s1, s2, s3):
    wid = lax.axis_index("s") * _NC + lax.axis_index("c")
    pltpu.sync_copy(batch.at[pl.ds(wid * _BW, _BW)], bidx)
    pltpu.async_copy(h.at[bidx], nodes_v, sn)
    for g in range(_BW // 16):
        bv = bidx[pl.ds(g * 16, 16)]
        for l in range(16):
            b = g * 16 + l
            pltpu.async_copy(adj.at[pl.ds(bv[l], 1)],
                             nidx.at[pl.ds(b, 1)], sa)
    pltpu.make_async_copy(adj.at[pl.ds(0, _BW)], nidx, sa).wait()

    bufs = (nr0, nr1, nr2, nr3)
    sems = (s0, s1, s2, s3)
    for p in range(_NBUF):
        pltpu.async_copy(h.at[nidx.at[p]], bufs[p], sems[p])

    def do_elem(b, buf, sem):
        pltpu.make_async_copy(h.at[pl.ds(0, _DEG)], buf, sem).wait()
        accs = _acc_rows(buf, 0, _DEG)
        for v in range(8):
            out_v[b, pl.ds(16 * v, 16)] = accs[v] * (1.0 / _DEG)

        @pl.when(b < _BW - _NBUF)
        def _():
            pltpu.async_copy(h.at[nidx.at[b + _NBUF]], buf, sem)

    def loop(i, carry):
        for p in range(_NBUF):
            do_elem(i * _NBUF + p, bufs[p], sems[p])
        return carry

    lax.fori_loop(0, _BW // _NBUF, loop, 0)
    pltpu.make_async_copy(h.at[pl.ds(0, _BW)], nodes_v, sn).wait()
    pltpu.sync_copy(nodes_v, node_out.at[pl.ds(wid * _BW, _BW)])
    pltpu.sync_copy(out_v, neigh_out.at[pl.ds(wid * _BW, _BW)])


_layer2_call = functools.partial(
    pl.kernel,
    out_type=(
        jax.ShapeDtypeStruct((_B, _H), jnp.float32),
        jax.ShapeDtypeStruct((_B, _H), jnp.float32),
    ),
    mesh=_mesh,
    scratch_types=[
        pltpu.VMEM((_BW,), jnp.int32),
        pltpu.VMEM((_BW, _DEG), jnp.int32),
        pltpu.VMEM((_BW, _H), jnp.float32),
        pltpu.VMEM((_DEG, _H), jnp.float32),
        pltpu.VMEM((_DEG, _H), jnp.float32),
        pltpu.VMEM((_DEG, _H), jnp.float32),
        pltpu.VMEM((_DEG, _H), jnp.float32),
        pltpu.VMEM((_BW, _H), jnp.float32),
        pltpu.SemaphoreType.DMA,
        pltpu.SemaphoreType.DMA,
        pltpu.SemaphoreType.DMA,
        pltpu.SemaphoreType.DMA,
        pltpu.SemaphoreType.DMA,
        pltpu.SemaphoreType.DMA,
    ],
)(_layer2_body)


def _m1_body(x_ref, m_ref, wa_ref, wb_ref, b_ref, o_ref):
    o_ref[...] = jnp.maximum(
        jnp.dot(x_ref[...], wa_ref[...], preferred_element_type=jnp.float32)
        + jnp.dot(m_ref[...], wb_ref[...], preferred_element_type=jnp.float32)
        + b_ref[...],
        0.0,
    )


def _m2_body(x_ref, m_ref, wa_ref, wb_ref, b_ref, o_ref):
    o_ref[...] = (
        jnp.dot(x_ref[...], wa_ref[...], preferred_element_type=jnp.float32)
        + jnp.dot(m_ref[...], wb_ref[...], preferred_element_type=jnp.float32)
        + b_ref[...]
    )


def _m1(feats_pad, nmean, w1aT, w1bT, b1):
    blk = 1024
    return pl.pallas_call(
        _m1_body,
        grid=(_NPAD // blk,),
        in_specs=[
            pl.BlockSpec((blk, _F), lambda i: (i, 0)),
            pl.BlockSpec((blk, _F), lambda i: (i, 0)),
            pl.BlockSpec((_F, _H), lambda i: (0, 0)),
            pl.BlockSpec((_F, _H), lambda i: (0, 0)),
            pl.BlockSpec((1, _H), lambda i: (0, 0)),
        ],
        out_specs=pl.BlockSpec((blk, _H), lambda i: (i, 0)),
        out_shape=jax.ShapeDtypeStruct((_NPAD, _H), jnp.float32),
    )(feats_pad, nmean, w1aT, w1bT, b1)


def _m2(node_emb, neigh_emb, w2aT, w2bT, b2):
    return pl.pallas_call(
        _m2_body,
        out_shape=jax.ShapeDtypeStruct((_B, _L), jnp.float32),
    )(node_emb, neigh_emb, w2aT, w2bT, b2)


def kernel(adj_list, feats, batch, W1, b1, W2, b2):
    adj_pad = jnp.concatenate([adj_list, adj_list[: _NPAD - _N]], axis=0)
    feats_pad = jnp.pad(feats, ((0, _NPAD - _N), (0, 0)))
    w1aT = W1[:, :_F].T
    w1bT = W1[:, _F:].T
    w2aT = W2[:, :_H].T
    w2bT = W2[:, _H:].T

    nmean = _nmean_call(adj_pad, feats)
    h_all = _m1(feats_pad, nmean, w1aT, w1bT, b1[None, :])
    node_emb, neigh_emb = _layer2_call(batch, adj_list, h_all)
    return _m2(node_emb, neigh_emb, w2aT, w2bT, b2[None, :])
